# Initial kernel scaffold; baseline (speedup 1.0000x reference)
import jax
import jax.numpy as jnp
from jax.experimental import pallas as pl


def kernel(atom_fea, nbr_fea, nbr_fea_idx, crystal_atom_idx, mask, w_emb, conv0_fc_w, conv0_fc_b, conv0_bn1_g, conv0_bn1_b, conv0_bn2_g, conv0_bn2_b, conv1_fc_w, conv1_fc_b, conv1_bn1_g, conv1_bn1_b, conv1_bn2_g, conv1_bn2_b, conv2_fc_w, conv2_fc_b, conv2_bn1_g, conv2_bn1_b, conv2_bn2_g, conv2_bn2_b, fc1_w, fc1_b, fc2_w, fc2_b, out_w, out_b):
    raise NotImplementedError("write your pallas kernel here")


# speedup vs baseline: 1.7897x; 1.7897x over previous
import functools

import jax
import jax.numpy as jnp
from jax import lax
from jax.experimental import pallas as pl
from jax.experimental.pallas import tpu as pltpu
from jax.experimental.pallas import tpu_sc as plsc

_F32 = jnp.float32


def _pick_k(n_chunks):
  for k in (10, 8, 5, 4, 2, 1):
    if n_chunks % k == 0 and n_chunks // k >= 32:
      return k
  for k in (10, 8, 5, 4, 2, 1):
    if n_chunks % k == 0:
      return k
  return 1


def _sc_gather(table, idx_flat):
  v, d = table.shape
  b = idx_flat.shape[0]
  n_chunks = b // 128
  k = _pick_k(n_chunks)
  c = k * 128
  n_sup = n_chunks // k
  n_loop = -(-n_sup // 32)
  idx3d = idx_flat.reshape(n_sup, k, 128)
  mesh = plsc.VectorSubcoreMesh(core_axis_name="c", subcore_axis_name="s")

  @functools.partial(
      pl.kernel,
      mesh=mesh,
      compiler_params=pltpu.CompilerParams(use_tc_tiling_on_sc=False),
      out_type=jax.ShapeDtypeStruct((b, d), _F32),
      scratch_types=[
          pltpu.VMEM((k, 128), jnp.int32),
          pltpu.VMEM((c, d), _F32),
          pltpu.SemaphoreType.DMA,
      ],
  )
  def gather(table_hbm, idx_hbm, out_hbm, idx_v, rows_v, sem):
    wid = lax.axis_index("s") * 2 + lax.axis_index("c")

    def body(s, carry):
      sup = s * 32 + wid

      @pl.when(sup < n_sup)
      def _():
        pltpu.sync_copy(idx_hbm.at[sup], idx_v)
        copies = [
            pltpu.async_copy(
                table_hbm.at[idx_v.at[j]],
                rows_v.at[pl.ds(j * 128, 128)],
                sem,
            )
            for j in range(k)
        ]
        for cp in copies:
          cp.wait()
        pltpu.sync_copy(rows_v, out_hbm.at[pl.ds(sup * c, c)])

      return carry

    lax.fori_loop(0, n_loop, body, 0)

  return gather(table, idx3d)


def _pick_block(n, cap):
  for r in range(min(n, cap), 0, -1):
    if n % r == 0 and (r % 8 == 0 or r == n):
      return r
  return n


def _embed_body(a_ref, m_ref, w_ref, masked_ref, af_ref):
  masked = a_ref[...] * m_ref[...]
  masked_ref[...] = masked
  af_ref[...] = jnp.dot(masked, w_ref[...], preferred_element_type=_F32)


def _embed(atom_fea, mask2d, w_embt):
  n, orig = atom_fea.shape
  af_dim = w_embt.shape[1]
  r = _pick_block(n, 2000)
  grid = (n // r,)
  return pl.pallas_call(
      _embed_body,
      grid=grid,
      in_specs=[
          pl.BlockSpec((r, orig), lambda i: (i, 0)),
          pl.BlockSpec((1, orig), lambda i: (0, 0)),
          pl.BlockSpec((orig, af_dim), lambda i: (0, 0)),
      ],
      out_specs=[
          pl.BlockSpec((r, orig), lambda i: (i, 0)),
          pl.BlockSpec((r, af_dim), lambda i: (i, 0)),
      ],
      out_shape=[
          jax.ShapeDtypeStruct((n, orig), _F32),
          jax.ShapeDtypeStruct((n, af_dim), _F32),
      ],
  )(atom_fea, mask2d, w_embt)


def _proj(g_ref, nbr_ref, af_ref, wn_ref, we_ref, ws_ref, scale):
  e = jnp.dot(g_ref[...], wn_ref[...] * scale, preferred_element_type=_F32)
  e = e + jnp.dot(nbr_ref[...], we_ref[...] * scale,
                  preferred_element_type=_F32)
  s = jnp.dot(af_ref[...], ws_ref[...] * scale, preferred_element_type=_F32)
  return e, s


def _stats_body(m, g_ref, nbr_ref, af_ref, wnf, wef, wsf, bf, wnc, wec, wsc,
                bc, out_ref):
  r = af_ref.shape[0]
  af_dim = af_ref.shape[1]
  ef, sf = _proj(g_ref, nbr_ref, af_ref, wnf, wef, wsf, 1.0)
  sf = sf + bf[...]
  ec, sc2 = _proj(g_ref, nbr_ref, af_ref, wnc, wec, wsc, 1.0)
  sc2 = sc2 + bc[...]
  gf = ef.reshape(r, m, af_dim) + sf[:, None, :]
  gc = ec.reshape(r, m, af_dim) + sc2[:, None, :]

  @pl.when(pl.program_id(0) == 0)
  def _():
    out_ref[...] = jnp.zeros_like(out_ref)

  out_ref[0:1, :] += jnp.sum(gf, axis=(0, 1))[None, :]
  out_ref[1:2, :] += jnp.sum(gf * gf, axis=(0, 1))[None, :]
  out_ref[2:3, :] += jnp.sum(gc, axis=(0, 1))[None, :]
  out_ref[3:4, :] += jnp.sum(gc * gc, axis=(0, 1))[None, :]


def _main_body(m, cnt, g_ref, nbr_ref, af_ref, wnf, wef, wsf, bf, wnc, wec,
               wsc, bc, st_ref, g1f, b1f, g1c, b1c, sum_ref, out2_ref):
  r = af_ref.shape[0]
  af_dim = af_ref.shape[1]
  inv = 1.0 / cnt
  mf = st_ref[0:1, :] * inv
  vf = st_ref[1:2, :] * inv - mf * mf
  scf = g1f[...] * lax.rsqrt(vf + 1e-5)
  shf = b1f[...] - mf * scf
  mc = st_ref[2:3, :] * inv
  vc = st_ref[3:4, :] * inv - mc * mc
  scc = g1c[...] * lax.rsqrt(vc + 1e-5)
  shc = b1c[...] - mc * scc

  ef, sf = _proj(g_ref, nbr_ref, af_ref, wnf, wef, wsf, scf)
  sf = sf + bf[...] * scf + shf
  ec, sc2 = _proj(g_ref, nbr_ref, af_ref, wnc, wec, wsc, scc)
  sc2 = sc2 + bc[...] * scc + shc

  filt = jax.nn.sigmoid(ef.reshape(r, m, af_dim) + sf[:, None, :])
  core = jax.nn.softplus(ec.reshape(r, m, af_dim) + sc2[:, None, :])
  sm = jnp.sum(filt * core, axis=1)
  sum_ref[...] = sm

  @pl.when(pl.program_id(0) == 0)
  def _():
    out2_ref[...] = jnp.zeros_like(out2_ref)

  out2_ref[0:1, :] += jnp.sum(sm, axis=0)[None, :]
  out2_ref[1:2, :] += jnp.sum(sm * sm, axis=0)[None, :]


def _update_body(cnt, af_ref, sm_ref, st_ref, g2, b2, out_ref):
  inv = 1.0 / cnt
  mu = st_ref[0:1, :] * inv
  var = st_ref[1:2, :] * inv - mu * mu
  s = g2[...] * lax.rsqrt(var + 1e-5)
  sh = b2[...] - mu * s
  out_ref[...] = jax.nn.softplus(af_ref[...] + sm_ref[...] * s + sh)


def _conv_layer(af, g_rows, nbr2, m, wnf, wef, wsf, bf, wnc, wec, wsc, bc,
                g1f, b1f, g1c, b1c, g2, b2):
  n, af_dim = af.shape
  r = _pick_block(n, 1000)
  grid = (n // r,)
  edge_r = r * m
  nbr_dim = nbr2.shape[1]

  g_spec = pl.BlockSpec((edge_r, af_dim), lambda i: (i, 0))
  nbr_spec = pl.BlockSpec((edge_r, nbr_dim), lambda i: (i, 0))
  af_spec = pl.BlockSpec((r, af_dim), lambda i: (i, 0))
  w_spec = pl.BlockSpec((af_dim, af_dim), lambda i: (0, 0))
  we_spec = pl.BlockSpec((nbr_dim, af_dim), lambda i: (0, 0))
  b_spec = pl.BlockSpec((1, af_dim), lambda i: (0, 0))
  st_spec = pl.BlockSpec((8, af_dim), lambda i: (0, 0))

  stats = pl.pallas_call(
      functools.partial(_stats_body, m),
      grid=grid,
      in_specs=[g_spec, nbr_spec, af_spec, w_spec, we_spec, w_spec, b_spec,
                w_spec, we_spec, w_spec, b_spec],
      out_specs=st_spec,
      out_shape=jax.ShapeDtypeStruct((8, af_dim), _F32),
  )(g_rows, nbr2, af, wnf, wef, wsf, bf, wnc, wec, wsc, bc)

  summed, st2 = pl.pallas_call(
      functools.partial(_main_body, m, float(n * m)),
      grid=grid,
      in_specs=[g_spec, nbr_spec, af_spec, w_spec, we_spec, w_spec, b_spec,
                w_spec, we_spec, w_spec, b_spec, st_spec, b_spec, b_spec,
                b_spec, b_spec],
      out_specs=[af_spec, st_spec],
      out_shape=[
          jax.ShapeDtypeStruct((n, af_dim), _F32),
          jax.ShapeDtypeStruct((8, af_dim), _F32),
      ],
  )(g_rows, nbr2, af, wnf, wef, wsf, bf, wnc, wec, wsc, bc, stats,
    g1f, b1f, g1c, b1c)

  r2 = _pick_block(n, 2000)
  return pl.pallas_call(
      functools.partial(_update_body, float(n)),
      grid=(n // r2,),
      in_specs=[
          pl.BlockSpec((r2, af_dim), lambda i: (i, 0)),
          pl.BlockSpec((r2, af_dim), lambda i: (i, 0)),
          st_spec,
          b_spec,
          b_spec,
      ],
      out_specs=pl.BlockSpec((r2, af_dim), lambda i: (i, 0)),
      out_shape=jax.ShapeDtypeStruct((n, af_dim), _F32),
  )(af, summed, st2, g2, b2)


def _readout_body(ncry, apc, g_ref, w1, b1, w2, b2, wo, bo, out_ref):
  g = g_ref[...]
  nrm = jnp.sqrt(jnp.sum(g * g, axis=1, keepdims=True))
  g = g / jnp.maximum(nrm, 1e-12)
  pooled = jnp.mean(g.reshape(ncry, apc, g.shape[1]), axis=1)
  h = jax.nn.softplus(
      jnp.dot(pooled, w1[...], preferred_element_type=_F32) + b1[...])
  h = jax.nn.softplus(
      jnp.dot(h, w2[...], preferred_element_type=_F32) + b2[...])
  out_ref[...] = jnp.dot(h, wo[...], preferred_element_type=_F32) + bo[...]


def _forward(atom_fea, nbr_fea, nbr_fea_idx, crystal_atom_idx, mask, w_emb,
             conv_params, fc1_w, fc1_b, fc2_w, fc2_b, out_w, out_b,
             gather_fn):
  n, orig = atom_fea.shape
  m = nbr_fea.shape[1]
  nbr_dim = nbr_fea.shape[2]
  af_dim = w_emb.shape[0]
  ncry, apc = crystal_atom_idx.shape

  idx_flat = nbr_fea_idx.astype(jnp.int32).reshape(-1)
  cidx_flat = crystal_atom_idx.astype(jnp.int32).reshape(-1)
  nbr2 = nbr_fea.reshape(n * m, nbr_dim)

  masked, af = _embed(atom_fea, mask.reshape(1, orig), w_emb.T)

  for (fw, fb, g1, b1, g2, b2) in conv_params:
    wsf = fw[0:af_dim, 0:af_dim].T
    wsc = fw[af_dim:2 * af_dim, 0:af_dim].T
    wnf = fw[0:af_dim, af_dim:2 * af_dim].T
    wnc = fw[af_dim:2 * af_dim, af_dim:2 * af_dim].T
    wef = fw[0:af_dim, 2 * af_dim:].T
    wec = fw[af_dim:2 * af_dim, 2 * af_dim:].T
    bf = fb[0:af_dim].reshape(1, af_dim)
    bc = fb[af_dim:].reshape(1, af_dim)
    g1f = g1[0:af_dim].reshape(1, af_dim)
    g1c = g1[af_dim:].reshape(1, af_dim)
    b1f = b1[0:af_dim].reshape(1, af_dim)
    b1c = b1[af_dim:].reshape(1, af_dim)
    g_rows = gather_fn(af, idx_flat)
    af = _conv_layer(af, g_rows, nbr2, m, wnf, wef, wsf, bf, wnc, wec, wsc,
                     bc, g1f, b1f, g1c, b1c, g2.reshape(1, af_dim),
                     b2.reshape(1, af_dim))

  g_cry = gather_fn(af, cidx_flat)

  wo_pad = jnp.pad(out_w.T, ((0, 0), (0, 128 - out_w.shape[0])))
  bo_pad = jnp.pad(out_b.reshape(1, -1), ((0, 0), (0, 128 - out_b.shape[0])))
  out = pl.pallas_call(
      functools.partial(_readout_body, ncry, apc),
      out_shape=jax.ShapeDtypeStruct((ncry, 128), _F32),
  )(g_cry, fc1_w.T, fc1_b.reshape(1, af_dim), fc2_w.T,
    fc2_b.reshape(1, af_dim), wo_pad, bo_pad)
  props = out[:, 0:1]
  return props, masked


def kernel(atom_fea, nbr_fea, nbr_fea_idx, crystal_atom_idx, mask, w_emb,
           conv0_fc_w, conv0_fc_b, conv0_bn1_g, conv0_bn1_b, conv0_bn2_g,
           conv0_bn2_b, conv1_fc_w, conv1_fc_b, conv1_bn1_g, conv1_bn1_b,
           conv1_bn2_g, conv1_bn2_b, conv2_fc_w, conv2_fc_b, conv2_bn1_g,
           conv2_bn1_b, conv2_bn2_g, conv2_bn2_b, fc1_w, fc1_b, fc2_w, fc2_b,
           out_w, out_b):
  conv_params = [
      (conv0_fc_w, conv0_fc_b, conv0_bn1_g, conv0_bn1_b, conv0_bn2_g,
       conv0_bn2_b),
      (conv1_fc_w, conv1_fc_b, conv1_bn1_g, conv1_bn1_b, conv1_bn2_g,
       conv1_bn2_b),
      (conv2_fc_w, conv2_fc_b, conv2_bn1_g, conv2_bn1_b, conv2_bn2_g,
       conv2_bn2_b),
  ]
  return _forward(atom_fea, nbr_fea, nbr_fea_idx, crystal_atom_idx, mask,
                  w_emb, conv_params, fc1_w, fc1_b, fc2_w, fc2_b, out_w,
                  out_b, _sc_gather)
